# baseline (speedup 1.0000x reference)
import jax
import jax.numpy as jnp
from jax.experimental import pallas as pl
from jax.experimental.pallas import tpu as pltpu

BN = 256
SUB = 2


def _fused_body(x_ref, wp_ref, bp_ref, g_ref, b_ref, w1_ref, b1_ref,
                w2_ref, b2_ref, out_ref):
    bs = BN // SUB
    w = wp_ref[...]
    for s in range(SUB):
        rows = pl.ds(s * bs, bs)
        x = jnp.maximum(x_ref[rows, :], 0.0).astype(jnp.bfloat16)
        y = jnp.dot(x, w, preferred_element_type=jnp.float32)
        y = y + bp_ref[...]
        mu = jnp.mean(y, axis=-1, keepdims=True)
        var = jnp.mean(jnp.square(y - mu), axis=-1, keepdims=True)
        y = (y - mu) / jnp.sqrt(var + 1e-5) * g_ref[...] + b_ref[...]
        h = jnp.dot(y.astype(jnp.bfloat16), w1_ref[...],
                    preferred_element_type=jnp.float32)
        h = h + b1_ref[...]
        h = 0.5 * h * (1.0 + jax.lax.erf(h * 0.7071067811865476))
        out_ref[rows, :] = (jnp.dot(h.astype(jnp.bfloat16), w2_ref[...],
                                    preferred_element_type=jnp.float32)
                            + b2_ref[...])


def kernel(mask_tokens, W_proj, b_proj, ln_g, ln_b, W1, b1, W2, b2):
    n, kdim = mask_tokens.shape
    dim = W_proj.shape[1]
    hidden = W1.shape[1]
    ncls = W2.shape[1]

    return pl.pallas_call(
        _fused_body,
        grid=(n // BN,),
        in_specs=[
            pl.BlockSpec((BN, kdim), lambda i: (i, 0)),
            pl.BlockSpec((kdim, dim), lambda i: (0, 0)),
            pl.BlockSpec((1, dim), lambda i: (0, 0)),
            pl.BlockSpec((1, dim), lambda i: (0, 0)),
            pl.BlockSpec((1, dim), lambda i: (0, 0)),
            pl.BlockSpec((dim, hidden), lambda i: (0, 0)),
            pl.BlockSpec((1, hidden), lambda i: (0, 0)),
            pl.BlockSpec((hidden, ncls), lambda i: (0, 0)),
            pl.BlockSpec((1, ncls), lambda i: (0, 0)),
        ],
        out_specs=pl.BlockSpec((BN, ncls), lambda i: (i, 0)),
        out_shape=jax.ShapeDtypeStruct((n, ncls), jnp.float32),
        compiler_params=pltpu.CompilerParams(
            dimension_semantics=("parallel",)),
    )(mask_tokens, W_proj.astype(jnp.bfloat16), b_proj.reshape(1, dim),
      ln_g.reshape(1, dim), ln_b.reshape(1, dim),
      W1.astype(jnp.bfloat16), b1.reshape(1, hidden),
      W2.astype(jnp.bfloat16), b2.reshape(1, ncls))


# speedup vs baseline: 1.0056x; 1.0056x over previous
import jax
import jax.numpy as jnp
from jax.experimental import pallas as pl
from jax.experimental.pallas import tpu as pltpu

BN = 2048
BK = 1024


def _fused_body(x_ref, wp_ref, bp_ref, g_ref, b_ref, w1_ref, b1_ref,
                w2_ref, b2_ref, out_ref, acc_ref):
    k = pl.program_id(1)
    nk = pl.num_programs(1)

    x = jnp.maximum(x_ref[...], 0.0).astype(jnp.bfloat16)
    d = jnp.dot(x, wp_ref[...], preferred_element_type=jnp.float32)

    @pl.when(k == 0)
    def _store():
        acc_ref[...] = d

    @pl.when(k != 0)
    def _accum():
        acc_ref[...] += d

    @pl.when(k == nk - 1)
    def _epilogue():
        y = acc_ref[...] + bp_ref[...]
        mu = jnp.mean(y, axis=-1, keepdims=True)
        var = jnp.mean(jnp.square(y - mu), axis=-1, keepdims=True)
        y = (y - mu) / jnp.sqrt(var + 1e-5) * g_ref[...] + b_ref[...]
        h = jnp.dot(y.astype(jnp.bfloat16), w1_ref[...],
                    preferred_element_type=jnp.float32)
        h = h + b1_ref[...]
        h = 0.5 * h * (1.0 + jax.lax.erf(h * 0.7071067811865476))
        out_ref[...] = (jnp.dot(h.astype(jnp.bfloat16), w2_ref[...],
                                preferred_element_type=jnp.float32)
                        + b2_ref[...])


def kernel(mask_tokens, W_proj, b_proj, ln_g, ln_b, W1, b1, W2, b2):
    n, kdim = mask_tokens.shape
    dim = W_proj.shape[1]
    hidden = W1.shape[1]
    ncls = W2.shape[1]

    grid = (n // BN, kdim // BK)
    return pl.pallas_call(
        _fused_body,
        grid=grid,
        in_specs=[
            pl.BlockSpec((BN, BK), lambda i, k: (i, k)),
            pl.BlockSpec((BK, dim), lambda i, k: (k, 0)),
            pl.BlockSpec((1, dim), lambda i, k: (0, 0)),
            pl.BlockSpec((1, dim), lambda i, k: (0, 0)),
            pl.BlockSpec((1, dim), lambda i, k: (0, 0)),
            pl.BlockSpec((dim, hidden), lambda i, k: (0, 0)),
            pl.BlockSpec((1, hidden), lambda i, k: (0, 0)),
            pl.BlockSpec((hidden, ncls), lambda i, k: (0, 0)),
            pl.BlockSpec((1, ncls), lambda i, k: (0, 0)),
        ],
        out_specs=pl.BlockSpec((BN, ncls), lambda i, k: (i, 0)),
        out_shape=jax.ShapeDtypeStruct((n, ncls), jnp.float32),
        scratch_shapes=[pltpu.VMEM((BN, dim), jnp.float32)],
        compiler_params=pltpu.CompilerParams(
            dimension_semantics=("parallel", "arbitrary")),
    )(mask_tokens, W_proj.astype(jnp.bfloat16), b_proj.reshape(1, dim),
      ln_g.reshape(1, dim), ln_b.reshape(1, dim),
      W1.astype(jnp.bfloat16), b1.reshape(1, hidden),
      W2.astype(jnp.bfloat16), b2.reshape(1, ncls))


# speedup vs baseline: 1.1723x; 1.1657x over previous
import jax
import jax.numpy as jnp
from jax.experimental import pallas as pl
from jax.experimental.pallas import tpu as pltpu

BN = 2048
BK = 1024


def _fused_body(x_ref, wp_ref, bp_ref, g_ref, b_ref, w1_ref, b1_ref,
                w2_ref, b2_ref, out_ref, acc_ref):
    k = pl.program_id(1)
    nk = pl.num_programs(1)

    @pl.when(k == 0)
    def _init():
        acc_ref[...] = jnp.zeros_like(acc_ref)

    x = jnp.maximum(x_ref[...], 0.0).astype(jnp.bfloat16)
    w = wp_ref[...].astype(jnp.bfloat16)
    acc_ref[...] += jnp.dot(x, w, preferred_element_type=jnp.float32)

    @pl.when(k == nk - 1)
    def _epilogue():
        y = acc_ref[...] + bp_ref[...]
        mu = jnp.mean(y, axis=-1, keepdims=True)
        var = jnp.mean(jnp.square(y - mu), axis=-1, keepdims=True)
        y = (y - mu) / jnp.sqrt(var + 1e-5) * g_ref[...] + b_ref[...]
        h = jnp.dot(y.astype(jnp.bfloat16), w1_ref[...].astype(jnp.bfloat16),
                    preferred_element_type=jnp.float32)
        h = h + b1_ref[...]
        h = 0.5 * h * (1.0 + jax.lax.erf(h * 0.7071067811865476))
        out_ref[...] = (jnp.dot(h.astype(jnp.bfloat16),
                                w2_ref[...].astype(jnp.bfloat16),
                                preferred_element_type=jnp.float32)
                        + b2_ref[...])


def kernel(mask_tokens, W_proj, b_proj, ln_g, ln_b, W1, b1, W2, b2):
    n, kdim = mask_tokens.shape
    dim = W_proj.shape[1]
    hidden = W1.shape[1]
    ncls = W2.shape[1]

    grid = (n // BN, kdim // BK)
    return pl.pallas_call(
        _fused_body,
        grid=grid,
        in_specs=[
            pl.BlockSpec((BN, BK), lambda i, k: (i, k)),
            pl.BlockSpec((BK, dim), lambda i, k: (k, 0)),
            pl.BlockSpec((1, dim), lambda i, k: (0, 0)),
            pl.BlockSpec((1, dim), lambda i, k: (0, 0)),
            pl.BlockSpec((1, dim), lambda i, k: (0, 0)),
            pl.BlockSpec((dim, hidden), lambda i, k: (0, 0)),
            pl.BlockSpec((1, hidden), lambda i, k: (0, 0)),
            pl.BlockSpec((hidden, ncls), lambda i, k: (0, 0)),
            pl.BlockSpec((1, ncls), lambda i, k: (0, 0)),
        ],
        out_specs=pl.BlockSpec((BN, ncls), lambda i, k: (i, 0)),
        out_shape=jax.ShapeDtypeStruct((n, ncls), jnp.float32),
        scratch_shapes=[pltpu.VMEM((BN, dim), jnp.float32)],
        compiler_params=pltpu.CompilerParams(
            dimension_semantics=("parallel", "arbitrary")),
    )(mask_tokens, W_proj, b_proj.reshape(1, dim), ln_g.reshape(1, dim),
      ln_b.reshape(1, dim), W1, b1.reshape(1, hidden), W2,
      b2.reshape(1, ncls))
